# Initial kernel scaffold; baseline (speedup 1.0000x reference)
import jax
import jax.numpy as jnp
from jax.experimental import pallas as pl


def kernel(nfeats, efeats, edge_index, W_msg1, b_msg1, W_apply1, b_apply1, W_msg2, b_msg2, W_apply2, b_apply2, W_msg3, b_msg3, W_apply3, b_apply3):
    raise NotImplementedError("write your pallas kernel here")


# speedup vs baseline: 2.1075x; 2.1075x over previous
import functools
import jax
import jax.numpy as jnp
from jax import lax
from jax.experimental import pallas as pl
from jax.experimental.pallas import tpu as pltpu
from jax.experimental.pallas import tpu_sc as plsc

N = 10000
E = 160000
NC = 2
NS = 16
L = 16
HW = 128

CH = 80
EPT = E // NS
NCH = EPT // CH
SS = 25
NSC = NCH // SS
RPT = N // NS

BN = 1000
GRID = N // BN


def _make_sc_msg(groups):
    mesh = plsc.VectorSubcoreMesh(core_axis_name="c", subcore_axis_name="s")

    @functools.partial(
        pl.kernel,
        out_type=jax.ShapeDtypeStruct((NC, NS, RPT, HW), jnp.float32),
        mesh=mesh,
        scratch_types=[
            pltpu.VMEM((SS, CH), jnp.int32),
            pltpu.VMEM((SS, CH), jnp.int32),
            pltpu.VMEM((SS, CH), jnp.float32),
            pltpu.VMEM((CH, HW), jnp.float32),
            pltpu.VMEM((HW,), jnp.float32),
            pltpu.VMEM_SHARED((N, HW), jnp.float32),
            pltpu.SemaphoreType.DMA,
        ],
    )
    def sc_msg(table, srcm, dstm, efm, wb, out,
               src_v, dst_v, ef_v, rows_v, wb_v, accum, sem):
        c = lax.axis_index("c")
        s = lax.axis_index("s")

        pltpu.sync_copy(wb.at[c], wb_v)

        def zfill(i, _):
            for g in range(HW // L):
                rows_v[i, pl.ds(g * L, L)] = jnp.zeros((L,), jnp.float32)
            return 0
        lax.fori_loop(0, CH, zfill, 0)
        for r in range(RPT // CH):
            pltpu.sync_copy(rows_v, accum.at[pl.ds(s * RPT + r * CH, CH)])
        pltpu.sync_copy(rows_v.at[pl.ds(0, RPT % CH)],
                        accum.at[pl.ds(s * RPT + (RPT // CH) * CH, RPT % CH)])
        plsc.subcore_barrier()

        def super_body(q, _):
            pltpu.sync_copy(srcm.at[s, q], src_v)
            pltpu.sync_copy(dstm.at[s, q], dst_v)
            pltpu.sync_copy(efm.at[s, q], ef_v)

            def chunk_body(k, _):
                pltpu.async_copy(table.at[c].at[src_v.at[k]], rows_v, sem).wait()

                def sub_body(t, _):
                    evec = ef_v[k, pl.ds(t * L, L)]
                    for j in range(L):
                        eb = jnp.full((L,), evec[j], jnp.float32)
                        i = t * L + j
                        for g in range(groups):
                            sl = pl.ds(g * L, L)
                            rows_v[i, sl] = jnp.maximum(
                                rows_v[i, sl] + eb * wb_v[sl], 0.0)
                    return 0
                lax.fori_loop(0, CH // L, sub_body, 0)

                pltpu.sync_copy(rows_v, accum.at[dst_v.at[k]], add=True)
                return 0
            lax.fori_loop(0, SS, chunk_body, 0)
            return 0
        lax.fori_loop(0, NSC, super_body, 0)
        plsc.subcore_barrier()

        pltpu.sync_copy(accum.at[pl.ds(s * RPT, RPT)], out.at[c, s])

    return sc_msg


_sc_msg_narrow = _make_sc_msg(1)
_sc_msg_wide = _make_sc_msg(8)


def _pad_cols(x, width):
    bn = x.shape[0]
    return jnp.concatenate(
        [x, jnp.zeros((bn, width - x.shape[1]), jnp.float32)], axis=1)


def _tc1_body(x_ref, w_ref, b_ref, p_ref, q_ref):
    acc = jnp.dot(x_ref[...], w_ref[...],
                  preferred_element_type=jnp.float32) + b_ref[...]
    p_ref[0, :, :] = _pad_cols(acc[:, :16], HW)
    p_ref[1, :, :] = _pad_cols(acc[:, 16:32], HW)
    q_ref[...] = acc[:, 32:64]


def _tc1(nfeats, wcat, bias):
    return pl.pallas_call(
        _tc1_body,
        grid=(GRID,),
        in_specs=[
            pl.BlockSpec((BN, 256), lambda i: (i, 0)),
            pl.BlockSpec((256, 64), lambda i: (0, 0)),
            pl.BlockSpec((1, 64), lambda i: (0, 0)),
        ],
        out_specs=[
            pl.BlockSpec((NC, BN, HW), lambda i: (0, i, 0)),
            pl.BlockSpec((BN, 32), lambda i: (i, 0)),
        ],
        out_shape=[
            jax.ShapeDtypeStruct((NC, N, HW), jnp.float32),
            jax.ShapeDtypeStruct((N, 32), jnp.float32),
        ],
    )(nfeats, wcat, bias)


def _make_tc_apply_next(hnw, ph, qw):
    nw = 2 * ph + qw

    def body(q_ref, hn_ref, wbt_ref, ba_ref, wn_ref, bn_ref, p_ref, qn_ref):
        h = q_ref[...]
        h = h + jnp.dot(hn_ref[0][:, :hnw], wbt_ref[0],
                        preferred_element_type=jnp.float32)
        h = h + jnp.dot(hn_ref[1][:, :hnw], wbt_ref[1],
                        preferred_element_type=jnp.float32)
        h = jnp.maximum(h + ba_ref[...], 0.0)
        acc = jnp.dot(h, wn_ref[...],
                      preferred_element_type=jnp.float32) + bn_ref[...]
        if ph < HW:
            p_ref[0, :, :] = _pad_cols(acc[:, :ph], HW)
            p_ref[1, :, :] = _pad_cols(acc[:, ph:2 * ph], HW)
        else:
            p_ref[0, :, :] = acc[:, :ph]
            p_ref[1, :, :] = acc[:, ph:2 * ph]
        qn_ref[...] = acc[:, 2 * ph:]

    def call(q, hn, wbt, ba, wn, bn):
        return pl.pallas_call(
            body,
            grid=(GRID,),
            in_specs=[
                pl.BlockSpec((BN, 32), lambda i: (i, 0)),
                pl.BlockSpec((NC, BN, HW), lambda i: (0, i, 0)),
                pl.BlockSpec((NC, hnw, 32), lambda i: (0, 0, 0)),
                pl.BlockSpec((1, 32), lambda i: (0, 0)),
                pl.BlockSpec((32, nw), lambda i: (0, 0)),
                pl.BlockSpec((1, nw), lambda i: (0, 0)),
            ],
            out_specs=[
                pl.BlockSpec((NC, BN, HW), lambda i: (0, i, 0)),
                pl.BlockSpec((BN, qw), lambda i: (i, 0)),
            ],
            out_shape=[
                jax.ShapeDtypeStruct((NC, N, HW), jnp.float32),
                jax.ShapeDtypeStruct((N, qw), jnp.float32),
            ],
        )(q, hn, wbt, ba, wn, bn)

    return call


_tc2 = _make_tc_apply_next(16, 16, 32)
_tc3 = _make_tc_apply_next(16, 128, 256)


def _tc4_body(r_ref, hn_ref, w_ref, b_ref, o_ref):
    acc = r_ref[...] + b_ref[...]
    acc = acc + jnp.dot(hn_ref[0], w_ref[0],
                        preferred_element_type=jnp.float32)
    acc = acc + jnp.dot(hn_ref[1], w_ref[1],
                        preferred_element_type=jnp.float32)
    acc = jnp.maximum(acc, 0.0)
    o_ref[...] = jnp.sum(acc, axis=1, keepdims=True)


def _tc4(r3, hn3, w3bt, ba3):
    return pl.pallas_call(
        _tc4_body,
        grid=(GRID,),
        in_specs=[
            pl.BlockSpec((BN, 256), lambda i: (i, 0)),
            pl.BlockSpec((NC, BN, HW), lambda i: (0, i, 0)),
            pl.BlockSpec((NC, 128, 256), lambda i: (0, 0, 0)),
            pl.BlockSpec((1, 256), lambda i: (0, 0)),
        ],
        out_specs=pl.BlockSpec((BN, 1), lambda i: (i, 0)),
        out_shape=jax.ShapeDtypeStruct((N, 1), jnp.float32),
    )(r3, hn3, w3bt, ba3)


def _pad_wb(wb_halves):
    k = wb_halves.shape[1]
    return jnp.concatenate(
        [wb_halves, jnp.zeros((NC, HW - k), jnp.float32)], axis=1)


def kernel(nfeats, efeats, edge_index,
           W_msg1, b_msg1, W_apply1, b_apply1,
           W_msg2, b_msg2, W_apply2, b_apply2,
           W_msg3, b_msg3, W_apply3, b_apply3):
    src = edge_index[0].astype(jnp.int32).reshape(NS, NSC, SS, CH)
    dst = edge_index[1].astype(jnp.int32).reshape(NS, NSC, SS, CH)
    efm = efeats.astype(jnp.float32).reshape(NS, NSC, SS, CH)

    w1cat = jnp.concatenate([W_msg1[:, :256].T, W_apply1[:, :256].T], axis=1)
    b1cat = jnp.concatenate([b_msg1, jnp.zeros((32,), jnp.float32)])[None, :]
    w1b = _pad_wb(W_msg1[:, 256].reshape(NC, 16))

    w1bt = jnp.stack([W_apply1[:, 256:].T[:16], W_apply1[:, 256:].T[16:]])
    ba1 = b_apply1[None, :]
    w2cat = jnp.concatenate([W_msg2[:, :32].T, W_apply2[:, :32].T], axis=1)
    b2cat = jnp.concatenate([b_msg2, jnp.zeros((32,), jnp.float32)])[None, :]
    w2b = _pad_wb(W_msg2[:, 32].reshape(NC, 16))

    w2bt = jnp.stack([W_apply2[:, 32:].T[:16], W_apply2[:, 32:].T[16:]])
    ba2 = b_apply2[None, :]
    w3cat = jnp.concatenate([W_msg3[:, :32].T, W_apply3[:, :32].T], axis=1)
    b3cat = jnp.concatenate([b_msg3, jnp.zeros((256,), jnp.float32)])[None, :]
    w3b = W_msg3[:, 32].reshape(NC, 128)

    w3bt = jnp.stack([W_apply3[:, 32:].T[:128], W_apply3[:, 32:].T[128:]])
    ba3 = b_apply3[None, :]

    p1, q1 = _tc1(nfeats, w1cat, b1cat)
    hn1 = _sc_msg_narrow(p1, src, dst, efm, w1b).reshape(NC, N, HW)
    p2, q2 = _tc2(q1, hn1, w1bt, ba1, w2cat, b2cat)
    hn2 = _sc_msg_narrow(p2, src, dst, efm, w2b).reshape(NC, N, HW)
    p3, r3 = _tc3(q2, hn2, w2bt, ba2, w3cat, b3cat)
    hn3 = _sc_msg_wide(p3, src, dst, efm, w3b).reshape(NC, N, HW)
    out = _tc4(r3, hn3, w3bt, ba3)
    return out.reshape(N)


# speedup vs baseline: 2.7995x; 1.3283x over previous
import functools
import jax
import jax.numpy as jnp
from jax import lax
from jax.experimental import pallas as pl
from jax.experimental.pallas import tpu as pltpu
from jax.experimental.pallas import tpu_sc as plsc

N = 10000
E = 160000
NC = 2
NS = 16
L = 16
HW = 128

CH = 80
EPT = E // NS
NCH = EPT // CH
SS = 25
NSC = NCH // SS
RPT = N // NS

BN = 1000
GRID = N // BN


def _make_sc_msg(groups):
    mesh = plsc.VectorSubcoreMesh(core_axis_name="c", subcore_axis_name="s")

    @functools.partial(
        pl.kernel,
        out_type=jax.ShapeDtypeStruct((NC, NS, RPT, HW), jnp.float32),
        mesh=mesh,
        scratch_types=[
            pltpu.VMEM((SS, CH), jnp.int32),
            pltpu.VMEM((SS, CH), jnp.int32),
            pltpu.VMEM((SS, CH), jnp.float32),
            pltpu.VMEM((2, CH, HW), jnp.float32),
            pltpu.VMEM((HW,), jnp.float32),
            pltpu.VMEM_SHARED((N, HW), jnp.float32),
            pltpu.SemaphoreType.DMA,
            pltpu.SemaphoreType.DMA,
        ],
    )
    def sc_msg(table, srcm, dstm, efm, wb, out,
               src_v, dst_v, ef_v, rows_v, wb_v, accum, sg0, sg1):
        c = lax.axis_index("c")
        s = lax.axis_index("s")
        sgs = (sg0, sg1)

        pltpu.sync_copy(wb.at[c], wb_v)

        def zfill(i, _):
            for g in range(HW // L):
                rows_v[0, i, pl.ds(g * L, L)] = jnp.zeros((L,), jnp.float32)
            return 0
        lax.fori_loop(0, CH, zfill, 0)
        for r in range(RPT // CH):
            pltpu.sync_copy(rows_v.at[0],
                            accum.at[pl.ds(s * RPT + r * CH, CH)])
        pltpu.sync_copy(rows_v.at[0].at[pl.ds(0, RPT % CH)],
                        accum.at[pl.ds(s * RPT + (RPT // CH) * CH, RPT % CH)])
        plsc.subcore_barrier()

        def fire_g(k, b):
            pltpu.async_copy(table.at[c].at[src_v.at[k]], rows_v.at[b],
                             sgs[b])

        def wait_g(k, b):
            pltpu.make_async_copy(table.at[c].at[src_v.at[k]], rows_v.at[b],
                                  sgs[b]).wait()

        def compute(k, b):
            def sub_body(t, _):
                evec = ef_v[k, pl.ds(t * L, L)]
                for j in range(L):
                    eb = jnp.full((L,), evec[j], jnp.float32)
                    i = t * L + j
                    for g in range(groups):
                        sl = pl.ds(g * L, L)
                        rows_v[b, i, sl] = jnp.maximum(
                            rows_v[b, i, sl] + eb * wb_v[sl], 0.0)
                return 0
            lax.fori_loop(0, CH // L, sub_body, 0)

        def scatter(k, b):
            pltpu.sync_copy(rows_v.at[b], accum.at[dst_v.at[k]], add=True)

        def super_body(q, _):
            pltpu.sync_copy(srcm.at[s, q], src_v)
            pltpu.sync_copy(dstm.at[s, q], dst_v)
            pltpu.sync_copy(efm.at[s, q], ef_v)

            fire_g(0, 0)

            def pair_body(p, _):
                ka = 2 * p
                kb = ka + 1
                wait_g(ka, 0)
                fire_g(kb, 1)
                compute(ka, 0)
                scatter(ka, 0)
                fire_g(ka + 2, 0)
                wait_g(kb, 1)
                compute(kb, 1)
                scatter(kb, 1)
                return 0
            lax.fori_loop(0, SS // 2, pair_body, 0)

            wait_g(SS - 1, 0)
            compute(SS - 1, 0)
            scatter(SS - 1, 0)
            return 0
        lax.fori_loop(0, NSC, super_body, 0)
        plsc.subcore_barrier()

        pltpu.sync_copy(accum.at[pl.ds(s * RPT, RPT)], out.at[c, s])

    return sc_msg


_sc_msg_narrow = _make_sc_msg(1)
_sc_msg_wide = _make_sc_msg(8)


def _pad_cols(x, width):
    bn = x.shape[0]
    return jnp.concatenate(
        [x, jnp.zeros((bn, width - x.shape[1]), jnp.float32)], axis=1)


def _tc1_body(x_ref, w_ref, b_ref, p_ref, q_ref):
    acc = jnp.dot(x_ref[...], w_ref[...],
                  preferred_element_type=jnp.float32) + b_ref[...]
    p_ref[0, :, :] = _pad_cols(acc[:, :16], HW)
    p_ref[1, :, :] = _pad_cols(acc[:, 16:32], HW)
    q_ref[...] = acc[:, 32:64]


def _tc1(nfeats, wcat, bias):
    return pl.pallas_call(
        _tc1_body,
        grid=(GRID,),
        in_specs=[
            pl.BlockSpec((BN, 256), lambda i: (i, 0)),
            pl.BlockSpec((256, 64), lambda i: (0, 0)),
            pl.BlockSpec((1, 64), lambda i: (0, 0)),
        ],
        out_specs=[
            pl.BlockSpec((NC, BN, HW), lambda i: (0, i, 0)),
            pl.BlockSpec((BN, 32), lambda i: (i, 0)),
        ],
        out_shape=[
            jax.ShapeDtypeStruct((NC, N, HW), jnp.float32),
            jax.ShapeDtypeStruct((N, 32), jnp.float32),
        ],
    )(nfeats, wcat, bias)


def _make_tc_apply_next(hnw, ph, qw):
    nw = 2 * ph + qw

    def body(q_ref, hn_ref, wbt_ref, ba_ref, wn_ref, bn_ref, p_ref, qn_ref):
        h = q_ref[...]
        h = h + jnp.dot(hn_ref[0][:, :hnw], wbt_ref[0],
                        preferred_element_type=jnp.float32)
        h = h + jnp.dot(hn_ref[1][:, :hnw], wbt_ref[1],
                        preferred_element_type=jnp.float32)
        h = jnp.maximum(h + ba_ref[...], 0.0)
        acc = jnp.dot(h, wn_ref[...],
                      preferred_element_type=jnp.float32) + bn_ref[...]
        if ph < HW:
            p_ref[0, :, :] = _pad_cols(acc[:, :ph], HW)
            p_ref[1, :, :] = _pad_cols(acc[:, ph:2 * ph], HW)
        else:
            p_ref[0, :, :] = acc[:, :ph]
            p_ref[1, :, :] = acc[:, ph:2 * ph]
        qn_ref[...] = acc[:, 2 * ph:]

    def call(q, hn, wbt, ba, wn, bn):
        return pl.pallas_call(
            body,
            grid=(GRID,),
            in_specs=[
                pl.BlockSpec((BN, 32), lambda i: (i, 0)),
                pl.BlockSpec((NC, BN, HW), lambda i: (0, i, 0)),
                pl.BlockSpec((NC, hnw, 32), lambda i: (0, 0, 0)),
                pl.BlockSpec((1, 32), lambda i: (0, 0)),
                pl.BlockSpec((32, nw), lambda i: (0, 0)),
                pl.BlockSpec((1, nw), lambda i: (0, 0)),
            ],
            out_specs=[
                pl.BlockSpec((NC, BN, HW), lambda i: (0, i, 0)),
                pl.BlockSpec((BN, qw), lambda i: (i, 0)),
            ],
            out_shape=[
                jax.ShapeDtypeStruct((NC, N, HW), jnp.float32),
                jax.ShapeDtypeStruct((N, qw), jnp.float32),
            ],
        )(q, hn, wbt, ba, wn, bn)

    return call


_tc2 = _make_tc_apply_next(16, 16, 32)
_tc3 = _make_tc_apply_next(16, 128, 256)


def _tc4_body(r_ref, hn_ref, w_ref, b_ref, o_ref):
    acc = r_ref[...] + b_ref[...]
    acc = acc + jnp.dot(hn_ref[0], w_ref[0],
                        preferred_element_type=jnp.float32)
    acc = acc + jnp.dot(hn_ref[1], w_ref[1],
                        preferred_element_type=jnp.float32)
    acc = jnp.maximum(acc, 0.0)
    o_ref[...] = jnp.sum(acc, axis=1, keepdims=True)


def _tc4(r3, hn3, w3bt, ba3):
    return pl.pallas_call(
        _tc4_body,
        grid=(GRID,),
        in_specs=[
            pl.BlockSpec((BN, 256), lambda i: (i, 0)),
            pl.BlockSpec((NC, BN, HW), lambda i: (0, i, 0)),
            pl.BlockSpec((NC, 128, 256), lambda i: (0, 0, 0)),
            pl.BlockSpec((1, 256), lambda i: (0, 0)),
        ],
        out_specs=pl.BlockSpec((BN, 1), lambda i: (i, 0)),
        out_shape=jax.ShapeDtypeStruct((N, 1), jnp.float32),
    )(r3, hn3, w3bt, ba3)


def _pad_wb(wb_halves):
    k = wb_halves.shape[1]
    return jnp.concatenate(
        [wb_halves, jnp.zeros((NC, HW - k), jnp.float32)], axis=1)


def kernel(nfeats, efeats, edge_index,
           W_msg1, b_msg1, W_apply1, b_apply1,
           W_msg2, b_msg2, W_apply2, b_apply2,
           W_msg3, b_msg3, W_apply3, b_apply3):
    src = edge_index[0].astype(jnp.int32).reshape(NS, NSC, SS, CH)
    dst = edge_index[1].astype(jnp.int32).reshape(NS, NSC, SS, CH)
    efm = efeats.astype(jnp.float32).reshape(NS, NSC, SS, CH)

    w1cat = jnp.concatenate([W_msg1[:, :256].T, W_apply1[:, :256].T], axis=1)
    b1cat = jnp.concatenate([b_msg1, jnp.zeros((32,), jnp.float32)])[None, :]
    w1b = _pad_wb(W_msg1[:, 256].reshape(NC, 16))

    w1bt = jnp.stack([W_apply1[:, 256:].T[:16], W_apply1[:, 256:].T[16:]])
    ba1 = b_apply1[None, :]
    w2cat = jnp.concatenate([W_msg2[:, :32].T, W_apply2[:, :32].T], axis=1)
    b2cat = jnp.concatenate([b_msg2, jnp.zeros((32,), jnp.float32)])[None, :]
    w2b = _pad_wb(W_msg2[:, 32].reshape(NC, 16))

    w2bt = jnp.stack([W_apply2[:, 32:].T[:16], W_apply2[:, 32:].T[16:]])
    ba2 = b_apply2[None, :]
    w3cat = jnp.concatenate([W_msg3[:, :32].T, W_apply3[:, :32].T], axis=1)
    b3cat = jnp.concatenate([b_msg3, jnp.zeros((256,), jnp.float32)])[None, :]
    w3b = W_msg3[:, 32].reshape(NC, 128)

    w3bt = jnp.stack([W_apply3[:, 32:].T[:128], W_apply3[:, 32:].T[128:]])
    ba3 = b_apply3[None, :]

    p1, q1 = _tc1(nfeats, w1cat, b1cat)
    hn1 = _sc_msg_narrow(p1, src, dst, efm, w1b).reshape(NC, N, HW)
    p2, q2 = _tc2(q1, hn1, w1bt, ba1, w2cat, b2cat)
    hn2 = _sc_msg_narrow(p2, src, dst, efm, w2b).reshape(NC, N, HW)
    p3, r3 = _tc3(q2, hn2, w2bt, ba2, w3cat, b3cat)
    hn3 = _sc_msg_wide(p3, src, dst, efm, w3b).reshape(NC, N, HW)
    out = _tc4(r3, hn3, w3bt, ba3)
    return out.reshape(N)
